# baseline (speedup 1.0000x reference)
import jax
import jax.numpy as jnp
from jax.experimental import pallas as pl

HC = 64
NN = {"user": 100000, "item": 100000}
ETS = [("user", "rates", "item"), ("item", "rated_by", "user")]


def _copy_body(x_ref, o_ref):
    o_ref[...] = x_ref[...]


def kernel(emb_user, emb_item, params, edge_index_user_rates_item, edge_index_item_rated_by_user):
    ei = {
        ("user", "rates", "item"): edge_index_user_rates_item,
        ("item", "rated_by", "user"): edge_index_item_rated_by_user,
    }
    x = {"user": emb_user, "item": emb_item}
    for l in range(2):
        out = {k: jnp.zeros((NN[k], HC), jnp.float32) for k in x}
        for (src, rel, dst) in ETS:
            e = ei[(src, rel, dst)]
            src_idx, dst_idx = e[0], e[1]
            msg = x[src][src_idx]
            summed = jax.ops.segment_sum(msg, dst_idx, num_segments=NN[dst])
            cnt = jax.ops.segment_sum(jnp.ones((e.shape[1],), jnp.float32), dst_idx, num_segments=NN[dst])
            mean = summed / jnp.maximum(cnt, 1.0)[:, None]
            p = params["l%d_%s" % (l, rel)]
            o = mean @ p["W_l"] + p["b_l"] + x[dst] @ p["W_r"]
            out[dst] = out[dst] + o
        x = out
        if l < 1:
            x = {k: jax.nn.relu(v) for k, v in x.items()}
    xu = pl.pallas_call(
        _copy_body,
        grid=(50,),
        in_specs=[pl.BlockSpec((2000, HC), lambda i: (i, 0))],
        out_specs=pl.BlockSpec((2000, HC), lambda i: (i, 0)),
        out_shape=jax.ShapeDtypeStruct(x["user"].shape, x["user"].dtype),
    )(x["user"])
    return (xu, x["item"])


# speedup vs baseline: 3.1629x; 3.1629x over previous
import functools

import jax
import jax.numpy as jnp
from jax import lax
from jax.experimental import pallas as pl
from jax.experimental.pallas import tpu as pltpu
from jax.experimental.pallas import tpu_sc as plsc

HC = 64
NNODE = 100000
E = 1200000
NC, NS = 2, 16
CHUNK = 128
KCH = 8
BLK = CHUNK * KCH
EPAD = 1228800
NCHUNKS = EPAD // CHUNK
CHUNKS_PER_TILE = NCHUNKS // NS
BLOCKS_PER_TILE = CHUNKS_PER_TILE // KCH
ACC_ROWS = 100096
ROWS_PER_TILE = ACC_ROWS // NS
DUMMY = NNODE


def _sc_body(do_cnt, *refs):
    if do_cnt:
        (yf_u, yf_i, sq_u, dp_u, sq_i, dp_i, zeros_h, ones_h,
         sums_item, sums_user, cnt_item, cnt_user,
         acc, sbuf, dbuf, rows, sem) = refs
    else:
        (yf_u, yf_i, sq_u, dp_u, sq_i, dp_i, zeros_h, ones_h,
         sums_item, sums_user,
         acc, sbuf, dbuf, rows, sem) = refs

    c = lax.axis_index("c")
    s = lax.axis_index("s")
    my_rows = pl.ds(s * ROWS_PER_TILE, ROWS_PER_TILE)
    chunk0 = s * CHUNKS_PER_TILE

    def seg_pass(sq, dp, yf, out3, q):
        pltpu.sync_copy(zeros_h, acc.at[my_rows])
        plsc.subcore_barrier()

        def blk(b, carry):
            r0 = chunk0 + b * KCH
            pltpu.sync_copy(sq.at[q, pl.ds(r0, KCH)], sbuf)
            pltpu.sync_copy(dp.at[pl.ds(r0, KCH)], dbuf)
            descs = [
                pltpu.async_copy(yf.at[sbuf.at[j]],
                                 rows.at[pl.ds(j * CHUNK, CHUNK)], sem)
                for j in range(KCH)
            ]
            for d in descs:
                d.wait()
            for j in range(KCH):
                pltpu.sync_copy(rows.at[pl.ds(j * CHUNK, CHUNK)],
                                acc.at[dbuf.at[j]], add=True)
            return carry

        lax.fori_loop(0, BLOCKS_PER_TILE, blk, 0)
        plsc.subcore_barrier()
        pltpu.sync_copy(acc.at[my_rows], out3.at[q, my_rows])

    for p in range(2):
        q = c * 2 + p
        seg_pass(sq_u, dp_u, yf_u, sums_item, q)
        seg_pass(sq_i, dp_i, yf_i, sums_user, q)

    if do_cnt:
        def cnt_pass(dp, out2):
            pltpu.sync_copy(ones_h, rows)
            pltpu.sync_copy(zeros_h, acc.at[my_rows])
            plsc.subcore_barrier()

            def blk(b, carry):
                r0 = chunk0 + b * KCH
                pltpu.sync_copy(dp.at[pl.ds(r0, KCH)], dbuf)
                for j in range(KCH):
                    pltpu.sync_copy(rows.at[pl.ds(j * CHUNK, CHUNK)],
                                    acc.at[dbuf.at[j]], add=True)
                return carry

            lax.fori_loop(0, BLOCKS_PER_TILE, blk, 0)
            plsc.subcore_barrier()
            pltpu.sync_copy(acc.at[my_rows], out2.at[my_rows])

        @pl.when(c == 0)
        def _():
            cnt_pass(dp_u, cnt_item)

        @pl.when(c == 1)
        def _():
            cnt_pass(dp_i, cnt_user)


def _make_sc(do_cnt):
    outs = [jax.ShapeDtypeStruct((4, ACC_ROWS, 16), jnp.float32)] * 2
    if do_cnt:
        outs += [jax.ShapeDtypeStruct((ACC_ROWS, 16), jnp.float32)] * 2
    mesh = plsc.VectorSubcoreMesh(
        core_axis_name="c", subcore_axis_name="s",
        num_cores=NC, num_subcores=NS)
    return pl.kernel(
        functools.partial(_sc_body, do_cnt),
        out_type=tuple(outs),
        mesh=mesh,
        scratch_types=[
            pltpu.VMEM_SHARED((ACC_ROWS, 16), jnp.float32),
            pltpu.VMEM((KCH, CHUNK), jnp.int32),
            pltpu.VMEM((KCH, CHUNK), jnp.int32),
            pltpu.VMEM((BLK, 16), jnp.float32),
            pltpu.SemaphoreType.DMA,
        ],
        compiler_params=pltpu.CompilerParams(use_tc_tiling_on_sc=False),
    )


_sc_l0 = _make_sc(True)
_sc_l1 = _make_sc(False)


def _mm_body(x_ref, w_ref, o_ref):
    o_ref[...] = jnp.dot(x_ref[...], w_ref[...],
                         preferred_element_type=jnp.float32)


def _mm(x, w):
    R = 2000
    return pl.pallas_call(
        _mm_body,
        grid=(NNODE // R,),
        in_specs=[pl.BlockSpec((R, HC), lambda i: (i, 0)),
                  pl.BlockSpec((HC, HC), lambda i: (0, 0))],
        out_specs=pl.BlockSpec((R, HC), lambda i: (i, 0)),
        out_shape=jax.ShapeDtypeStruct((NNODE, HC), jnp.float32),
    )(x, w)


def _post_body(relu, s_ref, c_ref, x_ref, wr_ref, b_ref, o_ref):
    sm = s_ref[...]
    m = jnp.concatenate([sm[0], sm[1], sm[2], sm[3]], axis=1)
    cnt = c_ref[...][:, 0:1]
    mean = m / jnp.maximum(cnt, 1.0)
    o = mean + b_ref[...] + jnp.dot(x_ref[...], wr_ref[...],
                                    preferred_element_type=jnp.float32)
    if relu:
        o = jnp.maximum(o, 0.0)
    o_ref[...] = o


def _post(sums, cnt, x, wr, b, relu):
    R = 2000
    return pl.pallas_call(
        functools.partial(_post_body, relu),
        grid=(NNODE // R,),
        in_specs=[pl.BlockSpec((4, R, 16), lambda i: (0, i, 0)),
                  pl.BlockSpec((R, 16), lambda i: (i, 0)),
                  pl.BlockSpec((R, HC), lambda i: (i, 0)),
                  pl.BlockSpec((HC, HC), lambda i: (0, 0)),
                  pl.BlockSpec((1, HC), lambda i: (0, 0))],
        out_specs=pl.BlockSpec((R, HC), lambda i: (i, 0)),
        out_shape=jax.ShapeDtypeStruct((NNODE, HC), jnp.float32),
    )(sums, cnt, x, wr, b)


def _prep(ei):
    src, dst = ei[0], ei[1]
    srcp = jnp.concatenate([src, jnp.zeros((EPAD - E,), jnp.int32)])
    dstp = jnp.concatenate([dst, jnp.full((EPAD - E,), DUMMY, jnp.int32)])
    srcq = (srcp * 4)[None, :] + jnp.arange(4, dtype=jnp.int32)[:, None]
    return srcq.reshape(4, NCHUNKS, CHUNK), dstp.reshape(NCHUNKS, CHUNK)


def kernel(emb_user, emb_item, params, edge_index_user_rates_item,
           edge_index_item_rated_by_user):
    sq_u, dp_u = _prep(edge_index_user_rates_item)
    sq_i, dp_i = _prep(edge_index_item_rated_by_user)
    zeros_h = jnp.zeros((ROWS_PER_TILE, 16), jnp.float32)
    ones_h = jnp.ones((BLK, 16), jnp.float32)

    xu, xi = emb_user, emb_item
    cnt_item = cnt_user = None
    for l in range(2):
        pu = params["l%d_rates" % l]
        pi = params["l%d_rated_by" % l]
        y_u = _mm(xu, pu["W_l"]).reshape(4 * NNODE, 16)
        y_i = _mm(xi, pi["W_l"]).reshape(4 * NNODE, 16)
        if l == 0:
            sums_item, sums_user, cnt_item, cnt_user = _sc_l0(
                y_u, y_i, sq_u, dp_u, sq_i, dp_i, zeros_h, ones_h)
        else:
            sums_item, sums_user = _sc_l1(
                y_u, y_i, sq_u, dp_u, sq_i, dp_i, zeros_h, ones_h)
        new_xi = _post(sums_item, cnt_item, xi, pu["W_r"],
                       pu["b_l"].reshape(1, HC), relu=(l == 0))
        new_xu = _post(sums_user, cnt_user, xu, pi["W_r"],
                       pi["b_l"].reshape(1, HC), relu=(l == 0))
        xu, xi = new_xu, new_xi
    return (xu, xi)


# speedup vs baseline: 3.2572x; 1.0298x over previous
import functools

import jax
import jax.numpy as jnp
from jax import lax
from jax.experimental import pallas as pl
from jax.experimental.pallas import tpu as pltpu
from jax.experimental.pallas import tpu_sc as plsc

HC = 64
NNODE = 100000
E = 1200000
NC, NS = 2, 16
BLK = 1024
EPAD = 1228800
EDGES_PER_TILE = EPAD // NS
BLOCKS_PER_TILE = EDGES_PER_TILE // BLK
ACC_ROWS = 100096
ROWS_PER_TILE = ACC_ROWS // NS
DUMMY = NNODE


def _sc_body(do_cnt, *refs):
    if do_cnt:
        (yf_u, yf_i, sq_u, dp_u, sq_i, dp_i, zeros_h, ones_h,
         sums_item, sums_user, cnt_item, cnt_user,
         acc, sbuf, dbuf, rows, sem) = refs
    else:
        (yf_u, yf_i, sq_u, dp_u, sq_i, dp_i, zeros_h, ones_h,
         sums_item, sums_user,
         acc, sbuf, dbuf, rows, sem) = refs

    c = lax.axis_index("c")
    s = lax.axis_index("s")
    my_rows = pl.ds(s * ROWS_PER_TILE, ROWS_PER_TILE)
    edge0 = s * EDGES_PER_TILE

    def seg_pass(sq, dp, yf, out3, q):
        pltpu.sync_copy(zeros_h, acc.at[my_rows])
        plsc.subcore_barrier()

        def blk(b, carry):
            e0 = edge0 + b * BLK
            pltpu.sync_copy(sq.at[q, pl.ds(e0, BLK)], sbuf)
            pltpu.sync_copy(dp.at[pl.ds(e0, BLK)], dbuf)
            pltpu.async_copy(yf.at[sbuf], rows, sem).wait()
            pltpu.sync_copy(rows, acc.at[dbuf], add=True)
            return carry

        lax.fori_loop(0, BLOCKS_PER_TILE, blk, 0)
        plsc.subcore_barrier()
        pltpu.sync_copy(acc.at[my_rows], out3.at[q, my_rows])

    for p in range(2):
        q = c * 2 + p
        seg_pass(sq_u, dp_u, yf_u, sums_item, q)
        seg_pass(sq_i, dp_i, yf_i, sums_user, q)

    if do_cnt:
        def cnt_pass(dp, out2):
            pltpu.sync_copy(ones_h, rows)
            pltpu.sync_copy(zeros_h, acc.at[my_rows])
            plsc.subcore_barrier()

            def blk(b, carry):
                e0 = edge0 + b * BLK
                pltpu.sync_copy(dp.at[pl.ds(e0, BLK)], dbuf)
                pltpu.sync_copy(rows, acc.at[dbuf], add=True)
                return carry

            lax.fori_loop(0, BLOCKS_PER_TILE, blk, 0)
            plsc.subcore_barrier()
            pltpu.sync_copy(acc.at[my_rows], out2.at[my_rows])

        @pl.when(c == 0)
        def _():
            cnt_pass(dp_u, cnt_item)

        @pl.when(c == 1)
        def _():
            cnt_pass(dp_i, cnt_user)


def _make_sc(do_cnt):
    outs = [jax.ShapeDtypeStruct((4, ACC_ROWS, 16), jnp.float32)] * 2
    if do_cnt:
        outs += [jax.ShapeDtypeStruct((ACC_ROWS, 16), jnp.float32)] * 2
    mesh = plsc.VectorSubcoreMesh(
        core_axis_name="c", subcore_axis_name="s",
        num_cores=NC, num_subcores=NS)
    return pl.kernel(
        functools.partial(_sc_body, do_cnt),
        out_type=tuple(outs),
        mesh=mesh,
        scratch_types=[
            pltpu.VMEM_SHARED((ACC_ROWS, 16), jnp.float32),
            pltpu.VMEM((BLK,), jnp.int32),
            pltpu.VMEM((BLK,), jnp.int32),
            pltpu.VMEM((BLK, 16), jnp.float32),
            pltpu.SemaphoreType.DMA,
        ],
        compiler_params=pltpu.CompilerParams(use_tc_tiling_on_sc=False),
    )


_sc_l0 = _make_sc(True)
_sc_l1 = _make_sc(False)


def _mm_body(x_ref, w_ref, o_ref):
    o_ref[...] = jnp.dot(x_ref[...], w_ref[...],
                         preferred_element_type=jnp.float32)


def _mm(x, w):
    R = 2000
    return pl.pallas_call(
        _mm_body,
        grid=(NNODE // R,),
        in_specs=[pl.BlockSpec((R, HC), lambda i: (i, 0)),
                  pl.BlockSpec((HC, HC), lambda i: (0, 0))],
        out_specs=pl.BlockSpec((R, HC), lambda i: (i, 0)),
        out_shape=jax.ShapeDtypeStruct((NNODE, HC), jnp.float32),
    )(x, w)


def _post_body(relu, s_ref, c_ref, x_ref, wr_ref, b_ref, o_ref):
    sm = s_ref[...]
    m = jnp.concatenate([sm[0], sm[1], sm[2], sm[3]], axis=1)
    cnt = c_ref[...][:, 0:1]
    mean = m / jnp.maximum(cnt, 1.0)
    o = mean + b_ref[...] + jnp.dot(x_ref[...], wr_ref[...],
                                    preferred_element_type=jnp.float32)
    if relu:
        o = jnp.maximum(o, 0.0)
    o_ref[...] = o


def _post(sums, cnt, x, wr, b, relu):
    R = 2000
    return pl.pallas_call(
        functools.partial(_post_body, relu),
        grid=(NNODE // R,),
        in_specs=[pl.BlockSpec((4, R, 16), lambda i: (0, i, 0)),
                  pl.BlockSpec((R, 16), lambda i: (i, 0)),
                  pl.BlockSpec((R, HC), lambda i: (i, 0)),
                  pl.BlockSpec((HC, HC), lambda i: (0, 0)),
                  pl.BlockSpec((1, HC), lambda i: (0, 0))],
        out_specs=pl.BlockSpec((R, HC), lambda i: (i, 0)),
        out_shape=jax.ShapeDtypeStruct((NNODE, HC), jnp.float32),
    )(sums, cnt, x, wr, b)


def _prep(ei):
    src, dst = ei[0], ei[1]
    srcp = jnp.concatenate([src, jnp.zeros((EPAD - E,), jnp.int32)])
    dstp = jnp.concatenate([dst, jnp.full((EPAD - E,), DUMMY, jnp.int32)])
    srcq = (srcp * 4)[None, :] + jnp.arange(4, dtype=jnp.int32)[:, None]
    return srcq, dstp


def kernel(emb_user, emb_item, params, edge_index_user_rates_item,
           edge_index_item_rated_by_user):
    sq_u, dp_u = _prep(edge_index_user_rates_item)
    sq_i, dp_i = _prep(edge_index_item_rated_by_user)
    zeros_h = jnp.zeros((ROWS_PER_TILE, 16), jnp.float32)
    ones_h = jnp.ones((BLK, 16), jnp.float32)

    xu, xi = emb_user, emb_item
    cnt_item = cnt_user = None
    for l in range(2):
        pu = params["l%d_rates" % l]
        pi = params["l%d_rated_by" % l]
        y_u = _mm(xu, pu["W_l"]).reshape(4 * NNODE, 16)
        y_i = _mm(xi, pi["W_l"]).reshape(4 * NNODE, 16)
        if l == 0:
            sums_item, sums_user, cnt_item, cnt_user = _sc_l0(
                y_u, y_i, sq_u, dp_u, sq_i, dp_i, zeros_h, ones_h)
        else:
            sums_item, sums_user = _sc_l1(
                y_u, y_i, sq_u, dp_u, sq_i, dp_i, zeros_h, ones_h)
        new_xi = _post(sums_item, cnt_item, xi, pu["W_r"],
                       pu["b_l"].reshape(1, HC), relu=(l == 0))
        new_xu = _post(sums_user, cnt_user, xu, pi["W_r"],
                       pi["b_l"].reshape(1, HC), relu=(l == 0))
        xu, xi = new_xu, new_xi
    return (xu, xi)


# speedup vs baseline: 3.5206x; 1.0809x over previous
import functools

import jax
import jax.numpy as jnp
from jax import lax
from jax.experimental import pallas as pl
from jax.experimental.pallas import tpu as pltpu
from jax.experimental.pallas import tpu_sc as plsc

HC = 64
NNODE = 100000
E = 1200000
NC, NS = 2, 16
BLK = 384
NSLOT = 4
EPAD = 1228800
NBT = EPAD // BLK
EDGES_PER_TILE = EPAD // NS
NB = EDGES_PER_TILE // BLK
ACC_ROWS = 100096
ROWS_PER_TILE = ACC_ROWS // NS
DUMMY = NNODE


def _sc_body(do_cnt, *refs):
    if do_cnt:
        (yf_u, yf_i, cq_u, cq_i, zeros_h, ones_h,
         sums_item, sums_user, cnt_item, cnt_user,
         acc, ci0, ci1, ci2, ci3, r0, r1, r2, r3,
         ga0, ga1, ga2, ga3, sb0, sb1, sb2, sb3) = refs
    else:
        (yf_u, yf_i, cq_u, cq_i, zeros_h, ones_h,
         sums_item, sums_user,
         acc, ci0, ci1, ci2, ci3, r0, r1, r2, r3,
         ga0, ga1, ga2, ga3, sb0, sb1, sb2, sb3) = refs
    cidx = (ci0, ci1, ci2, ci3)
    rows = (r0, r1, r2, r3)
    ga = (ga0, ga1, ga2, ga3)
    sb = (sb0, sb1, sb2, sb3)

    c = lax.axis_index("c")
    s = lax.axis_index("s")
    my_rows = pl.ds(s * ROWS_PER_TILE, ROWS_PER_TILE)
    blk0 = s * NB

    def seg_pass(cq, yf, out3, q):
        pltpu.sync_copy(zeros_h, acc.at[my_rows])
        plsc.subcore_barrier()

        def istart(b, k):
            pltpu.async_copy(cq.at[q, blk0 + b], cidx[k], ga[k])

        def gstart(k):
            pltpu.async_copy(yf.at[cidx[k].at[0]], rows[k], ga[k])

        def sstart(k):
            pltpu.async_copy(rows[k], acc.at[cidx[k].at[1]], sb[k], add=True)

        def iwait(k):
            pltpu.make_async_copy(cq.at[q, 0], cidx[k], ga[k]).wait()

        def gwait(k):
            pltpu.make_async_copy(yf.at[cidx[k].at[0]], rows[k], ga[k]).wait()

        def swait(k):
            pltpu.make_async_copy(rows[k], acc.at[cidx[k].at[1]],
                                  sb[k]).wait()

        istart(0, 0)

        def body(it, carry):
            for u in range(NSLOT):
                b = it * NSLOT + u
                k = u
                kp = (u + 1) % NSLOT
                km2 = (u + 2) % NSLOT
                iwait(k)
                gstart(k)

                @pl.when(b >= 2)
                def _():
                    gwait(km2)
                    sstart(km2)

                @pl.when(b >= 3)
                def _():
                    swait(kp)

                @pl.when(b + 1 < NB)
                def _():
                    istart(b + 1, kp)
            return carry

        lax.fori_loop(0, NB // NSLOT, body, 0)
        gwait((NB - 2) % NSLOT)
        sstart((NB - 2) % NSLOT)
        gwait((NB - 1) % NSLOT)
        sstart((NB - 1) % NSLOT)
        for b in (NB - 3, NB - 2, NB - 1):
            swait(b % NSLOT)
        plsc.subcore_barrier()
        pltpu.sync_copy(acc.at[my_rows], out3.at[q, my_rows])

    for p in range(2):
        q = c * 2 + p
        seg_pass(cq_u, yf_u, sums_item, q)
        seg_pass(cq_i, yf_i, sums_user, q)

    if do_cnt:
        def cnt_pass(cq, out2):
            pltpu.sync_copy(ones_h, rows[0])
            pltpu.sync_copy(zeros_h, acc.at[my_rows])
            plsc.subcore_barrier()

            def sstart(k):
                pltpu.async_copy(rows[0], acc.at[cidx[k].at[1]],
                                 sb[k], add=True)

            def swait(k):
                pltpu.make_async_copy(rows[0], acc.at[cidx[k].at[1]],
                                      sb[k]).wait()

            def body(it, carry):
                for u in range(NSLOT):
                    b = it * NSLOT + u
                    k = u

                    @pl.when(b >= NSLOT)
                    def _():
                        swait(k)

                    pltpu.sync_copy(cq.at[0, blk0 + b], cidx[k])
                    sstart(k)
                return carry

            lax.fori_loop(0, NB // NSLOT, body, 0)
            for k in range(NSLOT):
                swait(k)
            plsc.subcore_barrier()
            pltpu.sync_copy(acc.at[my_rows], out2.at[my_rows])

        @pl.when(c == 0)
        def _():
            cnt_pass(cq_u, cnt_item)

        @pl.when(c == 1)
        def _():
            cnt_pass(cq_i, cnt_user)


def _make_sc(do_cnt):
    outs = [jax.ShapeDtypeStruct((4, ACC_ROWS, 16), jnp.float32)] * 2
    if do_cnt:
        outs += [jax.ShapeDtypeStruct((ACC_ROWS, 16), jnp.float32)] * 2
    mesh = plsc.VectorSubcoreMesh(
        core_axis_name="c", subcore_axis_name="s",
        num_cores=NC, num_subcores=NS)
    return pl.kernel(
        functools.partial(_sc_body, do_cnt),
        out_type=tuple(outs),
        mesh=mesh,
        scratch_types=[
            pltpu.VMEM_SHARED((ACC_ROWS, 16), jnp.float32),
            pltpu.VMEM((2, BLK), jnp.int32),
            pltpu.VMEM((2, BLK), jnp.int32),
            pltpu.VMEM((2, BLK), jnp.int32),
            pltpu.VMEM((2, BLK), jnp.int32),
            pltpu.VMEM((BLK, 16), jnp.float32),
            pltpu.VMEM((BLK, 16), jnp.float32),
            pltpu.VMEM((BLK, 16), jnp.float32),
            pltpu.VMEM((BLK, 16), jnp.float32),
            pltpu.SemaphoreType.DMA,
            pltpu.SemaphoreType.DMA,
            pltpu.SemaphoreType.DMA,
            pltpu.SemaphoreType.DMA,
            pltpu.SemaphoreType.DMA,
            pltpu.SemaphoreType.DMA,
            pltpu.SemaphoreType.DMA,
            pltpu.SemaphoreType.DMA,
        ],
        compiler_params=pltpu.CompilerParams(use_tc_tiling_on_sc=False),
    )


_sc_l0 = _make_sc(True)
_sc_l1 = _make_sc(False)


def _mm_body(x_ref, w_ref, o_ref):
    o_ref[...] = jnp.dot(x_ref[...], w_ref[...],
                         preferred_element_type=jnp.float32)


def _mm(x, w):
    R = 2000
    return pl.pallas_call(
        _mm_body,
        grid=(NNODE // R,),
        in_specs=[pl.BlockSpec((R, HC), lambda i: (i, 0)),
                  pl.BlockSpec((HC, HC), lambda i: (0, 0))],
        out_specs=pl.BlockSpec((R, HC), lambda i: (i, 0)),
        out_shape=jax.ShapeDtypeStruct((NNODE, HC), jnp.float32),
    )(x, w)


def _post_body(relu, s_ref, c_ref, x_ref, wr_ref, b_ref, o_ref):
    sm = s_ref[...]
    m = jnp.concatenate([sm[0], sm[1], sm[2], sm[3]], axis=1)
    cnt = c_ref[...][:, 0:1]
    mean = m / jnp.maximum(cnt, 1.0)
    o = mean + b_ref[...] + jnp.dot(x_ref[...], wr_ref[...],
                                    preferred_element_type=jnp.float32)
    if relu:
        o = jnp.maximum(o, 0.0)
    o_ref[...] = o


def _post(sums, cnt, x, wr, b, relu):
    R = 2000
    return pl.pallas_call(
        functools.partial(_post_body, relu),
        grid=(NNODE // R,),
        in_specs=[pl.BlockSpec((4, R, 16), lambda i: (0, i, 0)),
                  pl.BlockSpec((R, 16), lambda i: (i, 0)),
                  pl.BlockSpec((R, HC), lambda i: (i, 0)),
                  pl.BlockSpec((HC, HC), lambda i: (0, 0)),
                  pl.BlockSpec((1, HC), lambda i: (0, 0))],
        out_specs=pl.BlockSpec((R, HC), lambda i: (i, 0)),
        out_shape=jax.ShapeDtypeStruct((NNODE, HC), jnp.float32),
    )(sums, cnt, x, wr, b)


def _prep(ei):
    src, dst = ei[0], ei[1]
    srcp = jnp.concatenate([src, jnp.zeros((EPAD - E,), jnp.int32)])
    dstp = jnp.concatenate([dst, jnp.full((EPAD - E,), DUMMY, jnp.int32)])
    srcq = (srcp * 4)[None, :] + jnp.arange(4, dtype=jnp.int32)[:, None]
    a = srcq.reshape(4, NBT, 1, BLK)
    b = jnp.broadcast_to(dstp.reshape(1, NBT, 1, BLK), (4, NBT, 1, BLK))
    return jnp.concatenate([a, b], axis=2)


def kernel(emb_user, emb_item, params, edge_index_user_rates_item,
           edge_index_item_rated_by_user):
    cq_u = _prep(edge_index_user_rates_item)
    cq_i = _prep(edge_index_item_rated_by_user)
    zeros_h = jnp.zeros((ROWS_PER_TILE, 16), jnp.float32)
    ones_h = jnp.ones((BLK, 16), jnp.float32)

    xu, xi = emb_user, emb_item
    cnt_item = cnt_user = None
    for l in range(2):
        pu = params["l%d_rates" % l]
        pi = params["l%d_rated_by" % l]
        y_u = _mm(xu, pu["W_l"]).reshape(4 * NNODE, 16)
        y_i = _mm(xi, pi["W_l"]).reshape(4 * NNODE, 16)
        if l == 0:
            sums_item, sums_user, cnt_item, cnt_user = _sc_l0(
                y_u, y_i, cq_u, cq_i, zeros_h, ones_h)
        else:
            sums_item, sums_user = _sc_l1(
                y_u, y_i, cq_u, cq_i, zeros_h, ones_h)
        new_xi = _post(sums_item, cnt_item, xi, pu["W_r"],
                       pu["b_l"].reshape(1, HC), relu=(l == 0))
        new_xu = _post(sums_user, cnt_user, xu, pi["W_r"],
                       pi["b_l"].reshape(1, HC), relu=(l == 0))
        xu, xi = new_xu, new_xi
    return (xu, xi)
